# baseline (speedup 1.0000x reference)
import functools

import jax
import jax.numpy as jnp
from jax import lax
from jax.experimental import pallas as pl
from jax.experimental.pallas import tpu as pltpu
from jax.experimental.pallas import tpu_sc as plsc

T = 2048
D = 768
F = 256
E = 64
NG = 8
GS = E // NG
TG = 4
RSF = 1.0

NEG = -1e30
BT = 256
BR = 128
NB = T * 2 // BR + E
S_MAX = NB * BR


def _sigmoid(x):
    return 1.0 / (1.0 + jnp.exp(-x))


def _routing_body(x_ref, gw_ref, b_ref, wsg_ref, wsu_ref, wsd_ref,
                  ids_ref, ws_ref, sh_ref, pos_ref, be_ref, idsc_ref):
    i = pl.program_id(0)

    @pl.when(i < T // BT)
    def _route():
        _route_step(x_ref, gw_ref, b_ref, wsg_ref, wsu_ref, wsd_ref,
                    ids_ref, ws_ref, sh_ref, idsc_ref, i)

    @pl.when(i == T // BT)
    def _do_plan():
        _plan_step(idsc_ref, pos_ref, be_ref)


def _route_step(x_ref, gw_ref, b_ref, wsg_ref, wsu_ref, wsd_ref,
                ids_ref, ws_ref, sh_ref, idsc_ref, i):
    x = x_ref[...]
    gw = gw_ref[...]
    logits = jax.lax.dot_general(
        x, gw, (((1,), (1,)), ((), ())), preferred_element_type=jnp.float32)
    scores = _sigmoid(logits)
    sfc = scores + b_ref[...]

    s3 = sfc.reshape(BT, NG, GS)
    gi = jax.lax.broadcasted_iota(jnp.int32, (BT, NG, GS), 2)
    m1 = jnp.max(s3, axis=-1)
    fidx = jnp.min(jnp.where(s3 >= m1[..., None], gi, GS), axis=-1)
    m2 = jnp.max(jnp.where(gi == fidx[..., None], NEG, s3), axis=-1)
    gsc = m1 + m2

    giota = jax.lax.broadcasted_iota(jnp.int32, (BT, NG), 1)
    gmask = jnp.zeros((BT, NG), jnp.float32)
    cur = gsc
    for _ in range(TG):
        mx = jnp.max(cur, axis=-1, keepdims=True)
        sel = jnp.min(jnp.where(cur >= mx, giota, NG), axis=-1, keepdims=True)
        hit = giota == sel
        gmask = jnp.where(hit, 1.0, gmask)
        cur = jnp.where(hit, NEG, cur)

    masked = jnp.where(gmask[..., None] > 0, s3, NEG).reshape(BT, E)
    eio = jax.lax.broadcasted_iota(jnp.int32, (BT, E), 1)
    mx1 = jnp.max(masked, axis=-1, keepdims=True)
    id1 = jnp.min(jnp.where(masked >= mx1, eio, E), axis=-1, keepdims=True)
    m2d = jnp.where(eio == id1, NEG, masked)
    mx2 = jnp.max(m2d, axis=-1, keepdims=True)
    id2 = jnp.min(jnp.where(m2d >= mx2, eio, E), axis=-1, keepdims=True)

    w1 = jnp.sum(jnp.where(eio == id1, scores, 0.0), axis=-1, keepdims=True)
    w2 = jnp.sum(jnp.where(eio == id2, scores, 0.0), axis=-1, keepdims=True)
    tot = w1 + w2 + 1e-20
    w1 = w1 / tot * RSF
    w2 = w2 / tot * RSF

    ids = jnp.concatenate([id1, id2], axis=1)
    ids_ref[...] = ids
    idsc_ref[pl.ds(i * BT, BT), :] = ids
    ws_ref[...] = jnp.concatenate([w1, w2], axis=1)

    gs = jnp.dot(x, wsg_ref[...], preferred_element_type=jnp.float32)
    us = jnp.dot(x, wsu_ref[...], preferred_element_type=jnp.float32)
    hs = _sigmoid(gs) * gs * us
    sh_ref[...] = jnp.dot(hs, wsd_ref[...], preferred_element_type=jnp.float32)


def _routing(x, gate_w, bias, wsg, wsu, wsd):
    nsteps = T // BT + 1
    return pl.pallas_call(
        _routing_body,
        grid=(nsteps,),
        in_specs=[
            pl.BlockSpec((BT, D), lambda i: (jnp.minimum(i, T // BT - 1), 0)),
            pl.BlockSpec((E, D), lambda i: (0, 0)),
            pl.BlockSpec((1, E), lambda i: (0, 0)),
            pl.BlockSpec((D, F), lambda i: (0, 0)),
            pl.BlockSpec((D, F), lambda i: (0, 0)),
            pl.BlockSpec((F, D), lambda i: (0, 0)),
        ],
        out_specs=[
            pl.BlockSpec((BT, 2), lambda i: (jnp.minimum(i, T // BT - 1), 0)),
            pl.BlockSpec((BT, 2), lambda i: (jnp.minimum(i, T // BT - 1), 0)),
            pl.BlockSpec((BT, D), lambda i: (jnp.minimum(i, T // BT - 1), 0)),
            pl.BlockSpec((T, 2), lambda i: (0, 0)),
            pl.BlockSpec((1, 128), lambda i: (0, 0)),
        ],
        out_shape=[
            jax.ShapeDtypeStruct((T, 2), jnp.int32),
            jax.ShapeDtypeStruct((T, 2), jnp.float32),
            jax.ShapeDtypeStruct((T, D), jnp.float32),
            jax.ShapeDtypeStruct((T, 2), jnp.int32),
            jax.ShapeDtypeStruct((1, 128), jnp.int32),
        ],
        scratch_shapes=[pltpu.VMEM((T, 2), jnp.int32)],
        compiler_params=pltpu.CompilerParams(
            dimension_semantics=("arbitrary",)),
    )(x, gate_w, bias, wsg, wsu, wsd)


CH = 512


def _plan_step(ids_ref, pos_ref, be_ref):
    ids = ids_ref[...]
    e0 = ids[:, 0:1]
    e1 = ids[:, 1:2]
    lane = jax.lax.broadcasted_iota(jnp.int32, (T, E), 1)
    oh0 = (lane == e0).astype(jnp.float32)
    oh1 = (lane == e1).astype(jnp.float32)

    ri = jax.lax.broadcasted_iota(jnp.int32, (CH, CH), 0)
    ci = jax.lax.broadcasted_iota(jnp.int32, (CH, CH), 1)
    Ls = (ci < ri).astype(jnp.float32)

    def prefix_excl(oh):
        outs = []
        carry = jnp.zeros((1, E), jnp.float32)
        for c in range(T // CH):
            blk = oh[c * CH:(c + 1) * CH, :]
            p = jax.lax.dot_general(
                Ls, blk, (((1,), (0,)), ((), ())),
                preferred_element_type=jnp.float32) + carry
            carry = carry + jnp.sum(blk, axis=0, keepdims=True)
            outs.append(p)
        return jnp.concatenate(outs, axis=0), carry

    P0, cnt0 = prefix_excl(oh0)
    P1, cnt1 = prefix_excl(oh1)
    P1 = P1 + cnt0
    counts = cnt0 + cnt1

    nb = jnp.floor((counts + (BR - 1)) / BR)
    li = jax.lax.broadcasted_iota(jnp.int32, (E, E), 0)
    lj = jax.lax.broadcasted_iota(jnp.int32, (E, E), 1)
    Uincl = (li <= lj).astype(jnp.float32)
    Uexcl = (li < lj).astype(jnp.float32)
    cum_incl = jnp.dot(nb, Uincl, preferred_element_type=jnp.float32)
    cum_excl = jnp.dot(nb, Uexcl, preferred_element_type=jnp.float32)
    off = cum_excl * BR

    rank0 = jnp.sum(oh0 * P0, axis=1, keepdims=True)
    rank1 = jnp.sum(oh1 * P1, axis=1, keepdims=True)
    base0 = jnp.sum(oh0 * off, axis=1, keepdims=True)
    base1 = jnp.sum(oh1 * off, axis=1, keepdims=True)
    pos0 = (base0 + rank0).astype(jnp.int32)
    pos1 = (base1 + rank1).astype(jnp.int32)
    pos_ref[...] = jnp.concatenate([pos0, pos1], axis=1)

    Ieye = (li == lj).astype(jnp.float32)
    cumT = jax.lax.dot_general(
        Ieye, cum_incl, (((1,), (1,)), ((), ())),
        preferred_element_type=jnp.float32)
    bio = jax.lax.broadcasted_iota(jnp.int32, (E, 128), 1)
    owner = jnp.sum((cumT.astype(jnp.int32) <= bio).astype(jnp.float32),
                    axis=0, keepdims=True)
    be_ref[...] = owner.astype(jnp.int32)


_SC_CORES = 2
_SC_SUBCORES = 16
NWORK = _SC_CORES * _SC_SUBCORES
TPW = T // NWORK


def _sc_dispatch(x, pos0, pos1):
    mesh = plsc.VectorSubcoreMesh(core_axis_name="c", subcore_axis_name="s")

    @functools.partial(
        pl.kernel,
        out_type=jax.ShapeDtypeStruct((S_MAX, D), jnp.float32),
        mesh=mesh,
        scratch_types=[
            pltpu.VMEM((TPW,), jnp.int32),
            pltpu.VMEM((TPW,), jnp.int32),
            pltpu.VMEM((TPW, D), jnp.float32),
            pltpu.SemaphoreType.DMA,
        ],
    )
    def k(x_hbm, p0_hbm, p1_hbm, xs_hbm, idx0_v, idx1_v, rows_v, sem):
        wid = lax.axis_index("s") * _SC_CORES + lax.axis_index("c")
        base = wid * TPW
        pltpu.sync_copy(p0_hbm.at[pl.ds(base, TPW)], idx0_v)
        pltpu.sync_copy(p1_hbm.at[pl.ds(base, TPW)], idx1_v)
        pltpu.sync_copy(x_hbm.at[pl.ds(base, TPW)], rows_v)
        pltpu.async_copy(rows_v, xs_hbm.at[idx0_v], sem).wait()
        pltpu.async_copy(rows_v, xs_hbm.at[idx1_v], sem).wait()

    return k(x, pos0, pos1)


def _grouped_body(be_ref, xs_ref, wg_ref, wu_ref, wd_ref, ys_ref):
    xs = xs_ref[...]
    g = jnp.dot(xs, wg_ref[0], preferred_element_type=jnp.float32)
    u = jnp.dot(xs, wu_ref[0], preferred_element_type=jnp.float32)
    h = _sigmoid(g) * g * u
    ys_ref[...] = jnp.dot(h, wd_ref[0], preferred_element_type=jnp.float32)


def _grouped(xs, be, Wg, Wu, Wd):
    grid_spec = pltpu.PrefetchScalarGridSpec(
        num_scalar_prefetch=1,
        grid=(NB,),
        in_specs=[
            pl.BlockSpec((BR, D), lambda b, be: (b, 0)),
            pl.BlockSpec((1, D, F),
                         lambda b, be: (jnp.minimum(be[b], E - 1), 0, 0)),
            pl.BlockSpec((1, D, F),
                         lambda b, be: (jnp.minimum(be[b], E - 1), 0, 0)),
            pl.BlockSpec((1, F, D),
                         lambda b, be: (jnp.minimum(be[b], E - 1), 0, 0)),
        ],
        out_specs=pl.BlockSpec((BR, D), lambda b, be: (b, 0)),
    )
    return pl.pallas_call(
        _grouped_body,
        grid_spec=grid_spec,
        out_shape=jax.ShapeDtypeStruct((S_MAX, D), jnp.float32),
        compiler_params=pltpu.CompilerParams(
            dimension_semantics=("arbitrary",)),
    )(be, xs, Wg, Wu, Wd)


def _sc_gather(ys, pos0, pos1):
    mesh = plsc.VectorSubcoreMesh(core_axis_name="c", subcore_axis_name="s")

    @functools.partial(
        pl.kernel,
        out_type=jax.ShapeDtypeStruct((2 * T, D), jnp.float32),
        mesh=mesh,
        scratch_types=[
            pltpu.VMEM((TPW,), jnp.int32),
            pltpu.VMEM((TPW, D), jnp.float32),
            pltpu.SemaphoreType.DMA,
        ],
    )
    def k(ys_hbm, p0_hbm, p1_hbm, yg_hbm, idx_v, rows_v, sem):
        wid = lax.axis_index("s") * _SC_CORES + lax.axis_index("c")
        base = wid * TPW
        pltpu.sync_copy(p0_hbm.at[pl.ds(base, TPW)], idx_v)
        pltpu.async_copy(ys_hbm.at[idx_v], rows_v, sem).wait()
        pltpu.sync_copy(rows_v, yg_hbm.at[pl.ds(base, TPW)])
        pltpu.sync_copy(p1_hbm.at[pl.ds(base, TPW)], idx_v)
        pltpu.async_copy(ys_hbm.at[idx_v], rows_v, sem).wait()
        pltpu.sync_copy(rows_v, yg_hbm.at[pl.ds(T + base, TPW)])

    return k(ys, pos0, pos1)


def _combine_body(sh_ref, yg0_ref, yg1_ref, ws_ref, o_ref):
    w0 = ws_ref[:, 0:1]
    w1 = ws_ref[:, 1:2]
    o_ref[...] = sh_ref[...] + w0 * yg0_ref[...] + w1 * yg1_ref[...]


def _combine(sh, yg, ws):
    return pl.pallas_call(
        _combine_body,
        grid=(T // BT,),
        in_specs=[
            pl.BlockSpec((BT, D), lambda i: (i, 0)),
            pl.BlockSpec((BT, D), lambda i: (i, 0)),
            pl.BlockSpec((BT, D), lambda i: (i + T // BT, 0)),
            pl.BlockSpec((BT, 2), lambda i: (i, 0)),
        ],
        out_specs=pl.BlockSpec((BT, D), lambda i: (i, 0)),
        out_shape=jax.ShapeDtypeStruct((T, D), jnp.float32),
    )(sh, yg, yg, ws)


def kernel(hidden_states, gate_w, e_score_correction_bias, Wg, Wu, Wd, Wsg, Wsu, Wsd):
    x = hidden_states
    bias = e_score_correction_bias.reshape(1, E)
    ids, ws, sh, pos, be = _routing(x, gate_w, bias, Wsg, Wsu, Wsd)
    pos0 = pos[:, 0]
    pos1 = pos[:, 1]
    be_vec = be[0, :]
    xs = _sc_dispatch(x, pos0, pos1)
    ys = _grouped(xs, be_vec, Wg, Wu, Wd)
    yg = _sc_gather(ys, pos0, pos1)
    return _combine(sh, yg, ws)


# speedup vs baseline: 1.1547x; 1.1547x over previous
import functools

import jax
import jax.numpy as jnp
from jax import lax
from jax.experimental import pallas as pl
from jax.experimental.pallas import tpu as pltpu
from jax.experimental.pallas import tpu_sc as plsc

T = 2048
D = 768
F = 256
E = 64
NG = 8
GS = E // NG
TG = 4
RSF = 1.0

NEG = -1e30
BT = 512
BR = 128
NB = T * 2 // BR + E
S_MAX = NB * BR


def _sigmoid(x):
    return 1.0 / (1.0 + jnp.exp(-x))


def _routing_body(x_ref, gw_ref, b_ref, wsg_ref, wsu_ref, wsd_ref,
                  ids_ref, ws_ref, sh_ref, pos_ref, be_ref, idsc_ref):
    i = pl.program_id(0)

    @pl.when(i < T // BT)
    def _route():
        _route_step(x_ref, gw_ref, b_ref, wsg_ref, wsu_ref, wsd_ref,
                    ids_ref, ws_ref, sh_ref, idsc_ref, i)

    @pl.when(i == T // BT)
    def _do_plan():
        _plan_step(idsc_ref, pos_ref, be_ref)


def _route_step(x_ref, gw_ref, b_ref, wsg_ref, wsu_ref, wsd_ref,
                ids_ref, ws_ref, sh_ref, idsc_ref, i):
    x = x_ref[...]
    gw = gw_ref[...]
    logits = jax.lax.dot_general(
        x, gw, (((1,), (1,)), ((), ())), preferred_element_type=jnp.float32)
    scores = _sigmoid(logits)
    sfc = scores + b_ref[...]

    s3 = sfc.reshape(BT, NG, GS)
    gi = jax.lax.broadcasted_iota(jnp.int32, (BT, NG, GS), 2)
    m1 = jnp.max(s3, axis=-1)
    fidx = jnp.min(jnp.where(s3 >= m1[..., None], gi, GS), axis=-1)
    m2 = jnp.max(jnp.where(gi == fidx[..., None], NEG, s3), axis=-1)
    gsc = m1 + m2

    giota = jax.lax.broadcasted_iota(jnp.int32, (BT, NG), 1)
    gmask = jnp.zeros((BT, NG), jnp.float32)
    cur = gsc
    for _ in range(TG):
        mx = jnp.max(cur, axis=-1, keepdims=True)
        sel = jnp.min(jnp.where(cur >= mx, giota, NG), axis=-1, keepdims=True)
        hit = giota == sel
        gmask = jnp.where(hit, 1.0, gmask)
        cur = jnp.where(hit, NEG, cur)

    masked = jnp.where(gmask[..., None] > 0, s3, NEG).reshape(BT, E)
    eio = jax.lax.broadcasted_iota(jnp.int32, (BT, E), 1)
    mx1 = jnp.max(masked, axis=-1, keepdims=True)
    id1 = jnp.min(jnp.where(masked >= mx1, eio, E), axis=-1, keepdims=True)
    m2d = jnp.where(eio == id1, NEG, masked)
    mx2 = jnp.max(m2d, axis=-1, keepdims=True)
    id2 = jnp.min(jnp.where(m2d >= mx2, eio, E), axis=-1, keepdims=True)

    w1 = jnp.sum(jnp.where(eio == id1, scores, 0.0), axis=-1, keepdims=True)
    w2 = jnp.sum(jnp.where(eio == id2, scores, 0.0), axis=-1, keepdims=True)
    tot = w1 + w2 + 1e-20
    w1 = w1 / tot * RSF
    w2 = w2 / tot * RSF

    ids = jnp.concatenate([id1, id2], axis=1)
    ids_ref[...] = ids
    idsc_ref[pl.ds(i * BT, BT), :] = ids
    ws_ref[...] = jnp.concatenate([w1, w2], axis=1)

    gs = jnp.dot(x, wsg_ref[...], preferred_element_type=jnp.float32)
    us = jnp.dot(x, wsu_ref[...], preferred_element_type=jnp.float32)
    hs = _sigmoid(gs) * gs * us
    sh_ref[...] = jnp.dot(hs, wsd_ref[...], preferred_element_type=jnp.float32)


def _routing(x, gate_w, bias, wsg, wsu, wsd):
    nsteps = T // BT + 1
    return pl.pallas_call(
        _routing_body,
        grid=(nsteps,),
        in_specs=[
            pl.BlockSpec((BT, D), lambda i: (jnp.minimum(i, T // BT - 1), 0)),
            pl.BlockSpec((E, D), lambda i: (0, 0)),
            pl.BlockSpec((1, E), lambda i: (0, 0)),
            pl.BlockSpec((D, F), lambda i: (0, 0)),
            pl.BlockSpec((D, F), lambda i: (0, 0)),
            pl.BlockSpec((F, D), lambda i: (0, 0)),
        ],
        out_specs=[
            pl.BlockSpec((BT, 2), lambda i: (jnp.minimum(i, T // BT - 1), 0)),
            pl.BlockSpec((BT, 2), lambda i: (jnp.minimum(i, T // BT - 1), 0)),
            pl.BlockSpec((BT, D), lambda i: (jnp.minimum(i, T // BT - 1), 0)),
            pl.BlockSpec((T, 2), lambda i: (0, 0)),
            pl.BlockSpec((1, 128), lambda i: (0, 0)),
        ],
        out_shape=[
            jax.ShapeDtypeStruct((T, 2), jnp.int32),
            jax.ShapeDtypeStruct((T, 2), jnp.float32),
            jax.ShapeDtypeStruct((T, D), jnp.float32),
            jax.ShapeDtypeStruct((T, 2), jnp.int32),
            jax.ShapeDtypeStruct((1, 128), jnp.int32),
        ],
        scratch_shapes=[pltpu.VMEM((T, 2), jnp.int32)],
        compiler_params=pltpu.CompilerParams(
            dimension_semantics=("arbitrary",)),
    )(x, gate_w, bias, wsg, wsu, wsd)


CH = 512


def _plan_step(ids_ref, pos_ref, be_ref):
    ids = ids_ref[...]
    e0 = ids[:, 0:1]
    e1 = ids[:, 1:2]
    lane = jax.lax.broadcasted_iota(jnp.int32, (T, E), 1)
    oh0 = (lane == e0).astype(jnp.float32)
    oh1 = (lane == e1).astype(jnp.float32)

    ri = jax.lax.broadcasted_iota(jnp.int32, (CH, CH), 0)
    ci = jax.lax.broadcasted_iota(jnp.int32, (CH, CH), 1)
    Ls = (ci < ri).astype(jnp.float32)

    def prefix_excl(oh):
        outs = []
        carry = jnp.zeros((1, E), jnp.float32)
        for c in range(T // CH):
            blk = oh[c * CH:(c + 1) * CH, :]
            p = jax.lax.dot_general(
                Ls, blk, (((1,), (0,)), ((), ())),
                preferred_element_type=jnp.float32) + carry
            carry = carry + jnp.sum(blk, axis=0, keepdims=True)
            outs.append(p)
        return jnp.concatenate(outs, axis=0), carry

    P0, cnt0 = prefix_excl(oh0)
    P1, cnt1 = prefix_excl(oh1)
    P1 = P1 + cnt0
    counts = cnt0 + cnt1

    nb = jnp.floor((counts + (BR - 1)) / BR)
    li = jax.lax.broadcasted_iota(jnp.int32, (E, E), 0)
    lj = jax.lax.broadcasted_iota(jnp.int32, (E, E), 1)
    Uincl = (li <= lj).astype(jnp.float32)
    Uexcl = (li < lj).astype(jnp.float32)
    cum_incl = jnp.dot(nb, Uincl, preferred_element_type=jnp.float32)
    cum_excl = jnp.dot(nb, Uexcl, preferred_element_type=jnp.float32)
    off = cum_excl * BR

    rank0 = jnp.sum(oh0 * P0, axis=1, keepdims=True)
    rank1 = jnp.sum(oh1 * P1, axis=1, keepdims=True)
    base0 = jnp.sum(oh0 * off, axis=1, keepdims=True)
    base1 = jnp.sum(oh1 * off, axis=1, keepdims=True)
    pos0 = (base0 + rank0).astype(jnp.int32)
    pos1 = (base1 + rank1).astype(jnp.int32)
    pos_ref[...] = jnp.concatenate([pos0, pos1], axis=1)

    Ieye = (li == lj).astype(jnp.float32)
    cumT = jax.lax.dot_general(
        Ieye, cum_incl, (((1,), (1,)), ((), ())),
        preferred_element_type=jnp.float32)
    bio = jax.lax.broadcasted_iota(jnp.int32, (E, 128), 1)
    owner = jnp.sum((cumT.astype(jnp.int32) <= bio).astype(jnp.float32),
                    axis=0, keepdims=True)
    be_ref[...] = owner.astype(jnp.int32)


_SC_CORES = 2
_SC_SUBCORES = 16
NWORK = _SC_CORES * _SC_SUBCORES
TPW = T // NWORK


def _sc_dispatch(x, pos0, pos1):
    mesh = plsc.VectorSubcoreMesh(core_axis_name="c", subcore_axis_name="s")

    @functools.partial(
        pl.kernel,
        out_type=jax.ShapeDtypeStruct((S_MAX, D), jnp.float32),
        mesh=mesh,
        scratch_types=[
            pltpu.VMEM((TPW,), jnp.int32),
            pltpu.VMEM((TPW,), jnp.int32),
            pltpu.VMEM((TPW, D), jnp.float32),
            pltpu.SemaphoreType.DMA,
        ],
    )
    def k(x_hbm, p0_hbm, p1_hbm, xs_hbm, idx0_v, idx1_v, rows_v, sem):
        wid = lax.axis_index("s") * _SC_CORES + lax.axis_index("c")
        base = wid * TPW
        pltpu.sync_copy(p0_hbm.at[pl.ds(base, TPW)], idx0_v)
        pltpu.sync_copy(p1_hbm.at[pl.ds(base, TPW)], idx1_v)
        pltpu.sync_copy(x_hbm.at[pl.ds(base, TPW)], rows_v)
        pltpu.async_copy(rows_v, xs_hbm.at[idx0_v], sem).wait()
        pltpu.async_copy(rows_v, xs_hbm.at[idx1_v], sem).wait()

    return k(x, pos0, pos1)


def _grouped_body(be_ref, xs_ref, wg_ref, wu_ref, wd_ref, ys_ref):
    b = pl.program_id(0)

    @pl.when(be_ref[b] < E)
    def _():
        xs = xs_ref[...]
        g = jnp.dot(xs, wg_ref[0], preferred_element_type=jnp.float32)
        u = jnp.dot(xs, wu_ref[0], preferred_element_type=jnp.float32)
        h = _sigmoid(g) * g * u
        ys_ref[...] = jnp.dot(h, wd_ref[0], preferred_element_type=jnp.float32)


def _grouped(xs, be, Wg, Wu, Wd):
    grid_spec = pltpu.PrefetchScalarGridSpec(
        num_scalar_prefetch=1,
        grid=(NB,),
        in_specs=[
            pl.BlockSpec((BR, D),
                         lambda b, be: (jnp.where(be[b] < E, b, 0), 0)),
            pl.BlockSpec((1, D, F),
                         lambda b, be: (jnp.minimum(be[b], E - 1), 0, 0)),
            pl.BlockSpec((1, D, F),
                         lambda b, be: (jnp.minimum(be[b], E - 1), 0, 0)),
            pl.BlockSpec((1, F, D),
                         lambda b, be: (jnp.minimum(be[b], E - 1), 0, 0)),
        ],
        out_specs=pl.BlockSpec((BR, D),
                               lambda b, be: (jnp.where(be[b] < E, b, NB), 0)),
    )
    return pl.pallas_call(
        _grouped_body,
        grid_spec=grid_spec,
        out_shape=jax.ShapeDtypeStruct((S_MAX + BR, D), jnp.float32),
        compiler_params=pltpu.CompilerParams(
            dimension_semantics=("arbitrary",)),
    )(be, xs, Wg, Wu, Wd)


def _sc_gather(ys, pos0, pos1):
    mesh = plsc.VectorSubcoreMesh(core_axis_name="c", subcore_axis_name="s")

    @functools.partial(
        pl.kernel,
        out_type=jax.ShapeDtypeStruct((2 * T, D), jnp.float32),
        mesh=mesh,
        scratch_types=[
            pltpu.VMEM((TPW,), jnp.int32),
            pltpu.VMEM((TPW, D), jnp.float32),
            pltpu.SemaphoreType.DMA,
        ],
    )
    def k(ys_hbm, p0_hbm, p1_hbm, yg_hbm, idx_v, rows_v, sem):
        wid = lax.axis_index("s") * _SC_CORES + lax.axis_index("c")
        base = wid * TPW
        pltpu.sync_copy(p0_hbm.at[pl.ds(base, TPW)], idx_v)
        pltpu.async_copy(ys_hbm.at[idx_v], rows_v, sem).wait()
        pltpu.sync_copy(rows_v, yg_hbm.at[pl.ds(base, TPW)])
        pltpu.sync_copy(p1_hbm.at[pl.ds(base, TPW)], idx_v)
        pltpu.async_copy(ys_hbm.at[idx_v], rows_v, sem).wait()
        pltpu.sync_copy(rows_v, yg_hbm.at[pl.ds(T + base, TPW)])

    return k(ys, pos0, pos1)


def _combine_body(sh_ref, yg0_ref, yg1_ref, ws_ref, o_ref):
    w0 = ws_ref[:, 0:1]
    w1 = ws_ref[:, 1:2]
    o_ref[...] = sh_ref[...] + w0 * yg0_ref[...] + w1 * yg1_ref[...]


def _combine(sh, yg, ws):
    return pl.pallas_call(
        _combine_body,
        grid=(T // BT,),
        in_specs=[
            pl.BlockSpec((BT, D), lambda i: (i, 0)),
            pl.BlockSpec((BT, D), lambda i: (i, 0)),
            pl.BlockSpec((BT, D), lambda i: (i + T // BT, 0)),
            pl.BlockSpec((BT, 2), lambda i: (i, 0)),
        ],
        out_specs=pl.BlockSpec((BT, D), lambda i: (i, 0)),
        out_shape=jax.ShapeDtypeStruct((T, D), jnp.float32),
    )(sh, yg, yg, ws)


def kernel(hidden_states, gate_w, e_score_correction_bias, Wg, Wu, Wd, Wsg, Wsu, Wsd):
    x = hidden_states
    bias = e_score_correction_bias.reshape(1, E)
    ids, ws, sh, pos, be = _routing(x, gate_w, bias, Wsg, Wsu, Wsd)
    pos0 = pos[:, 0]
    pos1 = pos[:, 1]
    be_vec = be[0, :]
    xs = _sc_dispatch(x, pos0, pos1)
    ys = _grouped(xs, be_vec, Wg, Wu, Wd)
    yg = _sc_gather(ys, pos0, pos1)
    return _combine(sh, yg, ws)


# speedup vs baseline: 1.1735x; 1.0163x over previous
import functools

import jax
import jax.numpy as jnp
from jax import lax
from jax.experimental import pallas as pl
from jax.experimental.pallas import tpu as pltpu
from jax.experimental.pallas import tpu_sc as plsc

T = 2048
D = 768
F = 256
E = 64
NG = 8
GS = E // NG
TG = 4
RSF = 1.0

NEG = -1e30
BT = 1024
BR = 128
NB = T * 2 // BR + E
S_MAX = NB * BR


def _sigmoid(x):
    return 1.0 / (1.0 + jnp.exp(-x))


def _routing_body(x_ref, gw_ref, b_ref, wsg_ref, wsu_ref, wsd_ref,
                  ids_ref, ws_ref, sh_ref, pos_ref, be_ref, idsc_ref):
    i = pl.program_id(0)

    @pl.when(i < T // BT)
    def _route():
        _route_step(x_ref, gw_ref, b_ref, wsg_ref, wsu_ref, wsd_ref,
                    ids_ref, ws_ref, sh_ref, idsc_ref, i)

    @pl.when(i == T // BT)
    def _do_plan():
        _plan_step(idsc_ref, pos_ref, be_ref)


def _route_step(x_ref, gw_ref, b_ref, wsg_ref, wsu_ref, wsd_ref,
                ids_ref, ws_ref, sh_ref, idsc_ref, i):
    x = x_ref[...]
    gw = gw_ref[...]
    logits = jax.lax.dot_general(
        x, gw, (((1,), (1,)), ((), ())), preferred_element_type=jnp.float32)
    scores = _sigmoid(logits)
    sfc = scores + b_ref[...]

    s3 = sfc.reshape(BT, NG, GS)
    gi = jax.lax.broadcasted_iota(jnp.int32, (BT, NG, GS), 2)
    m1 = jnp.max(s3, axis=-1)
    fidx = jnp.min(jnp.where(s3 >= m1[..., None], gi, GS), axis=-1)
    m2 = jnp.max(jnp.where(gi == fidx[..., None], NEG, s3), axis=-1)
    gsc = m1 + m2

    giota = jax.lax.broadcasted_iota(jnp.int32, (BT, NG), 1)
    gmask = jnp.zeros((BT, NG), jnp.float32)
    cur = gsc
    for _ in range(TG):
        mx = jnp.max(cur, axis=-1, keepdims=True)
        sel = jnp.min(jnp.where(cur >= mx, giota, NG), axis=-1, keepdims=True)
        hit = giota == sel
        gmask = jnp.where(hit, 1.0, gmask)
        cur = jnp.where(hit, NEG, cur)

    masked = jnp.where(gmask[..., None] > 0, s3, NEG).reshape(BT, E)
    eio = jax.lax.broadcasted_iota(jnp.int32, (BT, E), 1)
    mx1 = jnp.max(masked, axis=-1, keepdims=True)
    id1 = jnp.min(jnp.where(masked >= mx1, eio, E), axis=-1, keepdims=True)
    m2d = jnp.where(eio == id1, NEG, masked)
    mx2 = jnp.max(m2d, axis=-1, keepdims=True)
    id2 = jnp.min(jnp.where(m2d >= mx2, eio, E), axis=-1, keepdims=True)

    w1 = jnp.sum(jnp.where(eio == id1, scores, 0.0), axis=-1, keepdims=True)
    w2 = jnp.sum(jnp.where(eio == id2, scores, 0.0), axis=-1, keepdims=True)
    tot = w1 + w2 + 1e-20
    w1 = w1 / tot * RSF
    w2 = w2 / tot * RSF

    ids = jnp.concatenate([id1, id2], axis=1)
    ids_ref[...] = ids
    idsc_ref[pl.ds(i * BT, BT), :] = ids
    ws_ref[...] = jnp.concatenate([w1, w2], axis=1)

    gs = jnp.dot(x, wsg_ref[...], preferred_element_type=jnp.float32)
    us = jnp.dot(x, wsu_ref[...], preferred_element_type=jnp.float32)
    hs = _sigmoid(gs) * gs * us
    sh_ref[...] = jnp.dot(hs, wsd_ref[...], preferred_element_type=jnp.float32)


def _routing(x, gate_w, bias, wsg, wsu, wsd):
    nsteps = T // BT + 1
    return pl.pallas_call(
        _routing_body,
        grid=(nsteps,),
        in_specs=[
            pl.BlockSpec((BT, D), lambda i: (jnp.minimum(i, T // BT - 1), 0)),
            pl.BlockSpec((E, D), lambda i: (0, 0)),
            pl.BlockSpec((1, E), lambda i: (0, 0)),
            pl.BlockSpec((D, F), lambda i: (0, 0)),
            pl.BlockSpec((D, F), lambda i: (0, 0)),
            pl.BlockSpec((F, D), lambda i: (0, 0)),
        ],
        out_specs=[
            pl.BlockSpec((BT, 2), lambda i: (jnp.minimum(i, T // BT - 1), 0)),
            pl.BlockSpec((BT, 2), lambda i: (jnp.minimum(i, T // BT - 1), 0)),
            pl.BlockSpec((BT, D), lambda i: (jnp.minimum(i, T // BT - 1), 0)),
            pl.BlockSpec((T, 2), lambda i: (0, 0)),
            pl.BlockSpec((1, 128), lambda i: (0, 0)),
        ],
        out_shape=[
            jax.ShapeDtypeStruct((T, 2), jnp.int32),
            jax.ShapeDtypeStruct((T, 2), jnp.float32),
            jax.ShapeDtypeStruct((T, D), jnp.float32),
            jax.ShapeDtypeStruct((T, 2), jnp.int32),
            jax.ShapeDtypeStruct((1, 128), jnp.int32),
        ],
        scratch_shapes=[pltpu.VMEM((T, 2), jnp.int32)],
        compiler_params=pltpu.CompilerParams(
            dimension_semantics=("arbitrary",)),
    )(x, gate_w, bias, wsg, wsu, wsd)


CH = 512


def _plan_step(ids_ref, pos_ref, be_ref):
    ids = ids_ref[...]
    e0 = ids[:, 0:1]
    e1 = ids[:, 1:2]
    lane = jax.lax.broadcasted_iota(jnp.int32, (T, E), 1)
    oh0 = (lane == e0).astype(jnp.float32)
    oh1 = (lane == e1).astype(jnp.float32)

    ri = jax.lax.broadcasted_iota(jnp.int32, (CH, CH), 0)
    ci = jax.lax.broadcasted_iota(jnp.int32, (CH, CH), 1)
    Ls = (ci < ri).astype(jnp.float32)

    def prefix_excl(oh):
        outs = []
        carry = jnp.zeros((1, E), jnp.float32)
        for c in range(T // CH):
            blk = oh[c * CH:(c + 1) * CH, :]
            p = jax.lax.dot_general(
                Ls, blk, (((1,), (0,)), ((), ())),
                preferred_element_type=jnp.float32) + carry
            carry = carry + jnp.sum(blk, axis=0, keepdims=True)
            outs.append(p)
        return jnp.concatenate(outs, axis=0), carry

    P0, cnt0 = prefix_excl(oh0)
    P1, cnt1 = prefix_excl(oh1)
    P1 = P1 + cnt0
    counts = cnt0 + cnt1

    nb = jnp.floor((counts + (BR - 1)) / BR)
    li = jax.lax.broadcasted_iota(jnp.int32, (E, E), 0)
    lj = jax.lax.broadcasted_iota(jnp.int32, (E, E), 1)
    Uincl = (li <= lj).astype(jnp.float32)
    Uexcl = (li < lj).astype(jnp.float32)
    cum_incl = jnp.dot(nb, Uincl, preferred_element_type=jnp.float32)
    cum_excl = jnp.dot(nb, Uexcl, preferred_element_type=jnp.float32)
    off = cum_excl * BR

    rank0 = jnp.sum(oh0 * P0, axis=1, keepdims=True)
    rank1 = jnp.sum(oh1 * P1, axis=1, keepdims=True)
    base0 = jnp.sum(oh0 * off, axis=1, keepdims=True)
    base1 = jnp.sum(oh1 * off, axis=1, keepdims=True)
    pos0 = (base0 + rank0).astype(jnp.int32)
    pos1 = (base1 + rank1).astype(jnp.int32)
    pos_ref[...] = jnp.concatenate([pos0, pos1], axis=1)

    Ieye = (li == lj).astype(jnp.float32)
    cumT = jax.lax.dot_general(
        Ieye, cum_incl, (((1,), (1,)), ((), ())),
        preferred_element_type=jnp.float32)
    bio = jax.lax.broadcasted_iota(jnp.int32, (E, 128), 1)
    owner = jnp.sum((cumT.astype(jnp.int32) <= bio).astype(jnp.float32),
                    axis=0, keepdims=True)
    be_ref[...] = owner.astype(jnp.int32)


_SC_CORES = 2
_SC_SUBCORES = 16
NWORK = _SC_CORES * _SC_SUBCORES
TPW = T // NWORK


def _sc_dispatch(x, pos0, pos1):
    mesh = plsc.VectorSubcoreMesh(core_axis_name="c", subcore_axis_name="s")

    @functools.partial(
        pl.kernel,
        out_type=jax.ShapeDtypeStruct((S_MAX, D), jnp.float32),
        mesh=mesh,
        scratch_types=[
            pltpu.VMEM((TPW,), jnp.int32),
            pltpu.VMEM((TPW,), jnp.int32),
            pltpu.VMEM((TPW, D), jnp.float32),
            pltpu.SemaphoreType.DMA,
        ],
    )
    def k(x_hbm, p0_hbm, p1_hbm, xs_hbm, idx0_v, idx1_v, rows_v, sem):
        wid = lax.axis_index("s") * _SC_CORES + lax.axis_index("c")
        base = wid * TPW
        pltpu.sync_copy(p0_hbm.at[pl.ds(base, TPW)], idx0_v)
        pltpu.sync_copy(p1_hbm.at[pl.ds(base, TPW)], idx1_v)
        pltpu.sync_copy(x_hbm.at[pl.ds(base, TPW)], rows_v)
        pltpu.async_copy(rows_v, xs_hbm.at[idx0_v], sem).wait()
        pltpu.async_copy(rows_v, xs_hbm.at[idx1_v], sem).wait()

    return k(x, pos0, pos1)


def _grouped_body(be_ref, xs_ref, wg_ref, wu_ref, wd_ref, ys_ref):
    b = pl.program_id(0)

    @pl.when(be_ref[b] < E)
    def _():
        xs = xs_ref[...]
        g = jnp.dot(xs, wg_ref[0], preferred_element_type=jnp.float32)
        u = jnp.dot(xs, wu_ref[0], preferred_element_type=jnp.float32)
        h = _sigmoid(g) * g * u
        ys_ref[...] = jnp.dot(h, wd_ref[0], preferred_element_type=jnp.float32)


def _grouped(xs, be, Wg, Wu, Wd):
    grid_spec = pltpu.PrefetchScalarGridSpec(
        num_scalar_prefetch=1,
        grid=(NB,),
        in_specs=[
            pl.BlockSpec((BR, D),
                         lambda b, be: (jnp.where(be[b] < E, b, 0), 0)),
            pl.BlockSpec((1, D, F),
                         lambda b, be: (jnp.minimum(be[b], E - 1), 0, 0)),
            pl.BlockSpec((1, D, F),
                         lambda b, be: (jnp.minimum(be[b], E - 1), 0, 0)),
            pl.BlockSpec((1, F, D),
                         lambda b, be: (jnp.minimum(be[b], E - 1), 0, 0)),
        ],
        out_specs=pl.BlockSpec((BR, D),
                               lambda b, be: (jnp.where(be[b] < E, b, NB), 0)),
    )
    return pl.pallas_call(
        _grouped_body,
        grid_spec=grid_spec,
        out_shape=jax.ShapeDtypeStruct((S_MAX + BR, D), jnp.float32),
        compiler_params=pltpu.CompilerParams(
            dimension_semantics=("arbitrary",)),
    )(be, xs, Wg, Wu, Wd)


def _sc_gather(ys, pos0, pos1):
    mesh = plsc.VectorSubcoreMesh(core_axis_name="c", subcore_axis_name="s")

    @functools.partial(
        pl.kernel,
        out_type=jax.ShapeDtypeStruct((2 * T, D), jnp.float32),
        mesh=mesh,
        scratch_types=[
            pltpu.VMEM((TPW,), jnp.int32),
            pltpu.VMEM((TPW, D), jnp.float32),
            pltpu.SemaphoreType.DMA,
        ],
    )
    def k(ys_hbm, p0_hbm, p1_hbm, yg_hbm, idx_v, rows_v, sem):
        wid = lax.axis_index("s") * _SC_CORES + lax.axis_index("c")
        base = wid * TPW
        pltpu.sync_copy(p0_hbm.at[pl.ds(base, TPW)], idx_v)
        pltpu.async_copy(ys_hbm.at[idx_v], rows_v, sem).wait()
        pltpu.sync_copy(rows_v, yg_hbm.at[pl.ds(base, TPW)])
        pltpu.sync_copy(p1_hbm.at[pl.ds(base, TPW)], idx_v)
        pltpu.async_copy(ys_hbm.at[idx_v], rows_v, sem).wait()
        pltpu.sync_copy(rows_v, yg_hbm.at[pl.ds(T + base, TPW)])

    return k(ys, pos0, pos1)


def _combine_body(sh_ref, yg0_ref, yg1_ref, ws_ref, o_ref):
    w0 = ws_ref[:, 0:1]
    w1 = ws_ref[:, 1:2]
    o_ref[...] = sh_ref[...] + w0 * yg0_ref[...] + w1 * yg1_ref[...]


def _combine(sh, yg, ws):
    return pl.pallas_call(
        _combine_body,
        grid=(T // BT,),
        in_specs=[
            pl.BlockSpec((BT, D), lambda i: (i, 0)),
            pl.BlockSpec((BT, D), lambda i: (i, 0)),
            pl.BlockSpec((BT, D), lambda i: (i + T // BT, 0)),
            pl.BlockSpec((BT, 2), lambda i: (i, 0)),
        ],
        out_specs=pl.BlockSpec((BT, D), lambda i: (i, 0)),
        out_shape=jax.ShapeDtypeStruct((T, D), jnp.float32),
    )(sh, yg, yg, ws)


def kernel(hidden_states, gate_w, e_score_correction_bias, Wg, Wu, Wd, Wsg, Wsu, Wsd):
    x = hidden_states
    bias = e_score_correction_bias.reshape(1, E)
    ids, ws, sh, pos, be = _routing(x, gate_w, bias, Wsg, Wsu, Wsd)
    pos0 = pos[:, 0]
    pos1 = pos[:, 1]
    be_vec = be[0, :]
    xs = _sc_dispatch(x, pos0, pos1)
    ys = _grouped(xs, be_vec, Wg, Wu, Wd)
    yg = _sc_gather(ys, pos0, pos1)
    return _combine(sh, yg, ws)


# speedup vs baseline: 1.2108x; 1.0318x over previous
import functools

import jax
import jax.numpy as jnp
from jax import lax
from jax.experimental import pallas as pl
from jax.experimental.pallas import tpu as pltpu
from jax.experimental.pallas import tpu_sc as plsc

T = 2048
D = 768
F = 256
E = 64
NG = 8
GS = E // NG
TG = 4
RSF = 1.0

NEG = -1e30
BT = 1024
BR = 128
NB = T * 2 // BR + E
S_MAX = NB * BR


def _sigmoid(x):
    return 1.0 / (1.0 + jnp.exp(-x))


def _routing_body(x_ref, gw_ref, b_ref, wsg_ref, wsu_ref, wsd_ref,
                  ids_ref, ws_ref, sh_ref, pos_ref, be_ref, idsc_ref):
    i = pl.program_id(0)

    @pl.when(i < T // BT)
    def _route():
        _route_step(x_ref, gw_ref, b_ref, wsg_ref, wsu_ref, wsd_ref,
                    ids_ref, ws_ref, sh_ref, idsc_ref, i)

    @pl.when(i == T // BT)
    def _do_plan():
        _plan_step(idsc_ref, pos_ref, be_ref)


def _route_step(x_ref, gw_ref, b_ref, wsg_ref, wsu_ref, wsd_ref,
                ids_ref, ws_ref, sh_ref, idsc_ref, i):
    x = x_ref[...]
    gw = gw_ref[...]
    logits = jax.lax.dot_general(
        x, gw, (((1,), (1,)), ((), ())), preferred_element_type=jnp.float32)
    scores = _sigmoid(logits)
    sfc = scores + b_ref[...]

    s3 = sfc.reshape(BT, NG, GS)
    gi = jax.lax.broadcasted_iota(jnp.int32, (BT, NG, GS), 2)
    m1 = jnp.max(s3, axis=-1)
    fidx = jnp.min(jnp.where(s3 >= m1[..., None], gi, GS), axis=-1)
    m2 = jnp.max(jnp.where(gi == fidx[..., None], NEG, s3), axis=-1)
    gsc = m1 + m2

    giota = jax.lax.broadcasted_iota(jnp.int32, (BT, NG), 1)
    gmask = jnp.zeros((BT, NG), jnp.float32)
    cur = gsc
    for _ in range(TG):
        mx = jnp.max(cur, axis=-1, keepdims=True)
        sel = jnp.min(jnp.where(cur >= mx, giota, NG), axis=-1, keepdims=True)
        hit = giota == sel
        gmask = jnp.where(hit, 1.0, gmask)
        cur = jnp.where(hit, NEG, cur)

    masked = jnp.where(gmask[..., None] > 0, s3, NEG).reshape(BT, E)
    eio = jax.lax.broadcasted_iota(jnp.int32, (BT, E), 1)
    mx1 = jnp.max(masked, axis=-1, keepdims=True)
    id1 = jnp.min(jnp.where(masked >= mx1, eio, E), axis=-1, keepdims=True)
    m2d = jnp.where(eio == id1, NEG, masked)
    mx2 = jnp.max(m2d, axis=-1, keepdims=True)
    id2 = jnp.min(jnp.where(m2d >= mx2, eio, E), axis=-1, keepdims=True)

    w1 = jnp.sum(jnp.where(eio == id1, scores, 0.0), axis=-1, keepdims=True)
    w2 = jnp.sum(jnp.where(eio == id2, scores, 0.0), axis=-1, keepdims=True)
    tot = w1 + w2 + 1e-20
    w1 = w1 / tot * RSF
    w2 = w2 / tot * RSF

    ids = jnp.concatenate([id1, id2], axis=1)
    ids_ref[...] = ids
    idsc_ref[pl.ds(i * BT, BT), :] = ids
    ws_ref[...] = jnp.concatenate([w1, w2], axis=1)

    gs = jnp.dot(x, wsg_ref[...], preferred_element_type=jnp.float32)
    us = jnp.dot(x, wsu_ref[...], preferred_element_type=jnp.float32)
    hs = _sigmoid(gs) * gs * us
    sh_ref[...] = jnp.dot(hs, wsd_ref[...], preferred_element_type=jnp.float32)


def _routing(x, gate_w, bias, wsg, wsu, wsd):
    nsteps = T // BT + 1
    return pl.pallas_call(
        _routing_body,
        grid=(nsteps,),
        in_specs=[
            pl.BlockSpec((BT, D), lambda i: (jnp.minimum(i, T // BT - 1), 0)),
            pl.BlockSpec((E, D), lambda i: (0, 0)),
            pl.BlockSpec((1, E), lambda i: (0, 0)),
            pl.BlockSpec((D, F), lambda i: (0, 0)),
            pl.BlockSpec((D, F), lambda i: (0, 0)),
            pl.BlockSpec((F, D), lambda i: (0, 0)),
        ],
        out_specs=[
            pl.BlockSpec((BT, 2), lambda i: (jnp.minimum(i, T // BT - 1), 0)),
            pl.BlockSpec((BT, 2), lambda i: (jnp.minimum(i, T // BT - 1), 0)),
            pl.BlockSpec((BT, D), lambda i: (jnp.minimum(i, T // BT - 1), 0)),
            pl.BlockSpec((T, 2), lambda i: (0, 0)),
            pl.BlockSpec((1, 128), lambda i: (0, 0)),
        ],
        out_shape=[
            jax.ShapeDtypeStruct((T, 2), jnp.int32),
            jax.ShapeDtypeStruct((T, 2), jnp.float32),
            jax.ShapeDtypeStruct((T, D), jnp.float32),
            jax.ShapeDtypeStruct((T, 2), jnp.int32),
            jax.ShapeDtypeStruct((1, 128), jnp.int32),
        ],
        scratch_shapes=[pltpu.VMEM((T, 2), jnp.int32)],
        compiler_params=pltpu.CompilerParams(
            dimension_semantics=("arbitrary",)),
    )(x, gate_w, bias, wsg, wsu, wsd)


CH = 512


def _plan_step(ids_ref, pos_ref, be_ref):
    ids = ids_ref[...]
    e0 = ids[:, 0:1]
    e1 = ids[:, 1:2]
    lane = jax.lax.broadcasted_iota(jnp.int32, (T, E), 1)
    oh0 = (lane == e0).astype(jnp.float32)
    oh1 = (lane == e1).astype(jnp.float32)

    ri = jax.lax.broadcasted_iota(jnp.int32, (CH, CH), 0)
    ci = jax.lax.broadcasted_iota(jnp.int32, (CH, CH), 1)
    Ls = (ci < ri).astype(jnp.float32)

    def prefix_excl(oh):
        outs = []
        carry = jnp.zeros((1, E), jnp.float32)
        for c in range(T // CH):
            blk = oh[c * CH:(c + 1) * CH, :]
            p = jax.lax.dot_general(
                Ls, blk, (((1,), (0,)), ((), ())),
                preferred_element_type=jnp.float32) + carry
            carry = carry + jnp.sum(blk, axis=0, keepdims=True)
            outs.append(p)
        return jnp.concatenate(outs, axis=0), carry

    P0, cnt0 = prefix_excl(oh0)
    P1, cnt1 = prefix_excl(oh1)
    P1 = P1 + cnt0
    counts = cnt0 + cnt1

    nb = jnp.floor((counts + (BR - 1)) / BR)
    li = jax.lax.broadcasted_iota(jnp.int32, (E, E), 0)
    lj = jax.lax.broadcasted_iota(jnp.int32, (E, E), 1)
    Uincl = (li <= lj).astype(jnp.float32)
    Uexcl = (li < lj).astype(jnp.float32)
    cum_incl = jnp.dot(nb, Uincl, preferred_element_type=jnp.float32)
    cum_excl = jnp.dot(nb, Uexcl, preferred_element_type=jnp.float32)
    off = cum_excl * BR

    rank0 = jnp.sum(oh0 * P0, axis=1, keepdims=True)
    rank1 = jnp.sum(oh1 * P1, axis=1, keepdims=True)
    base0 = jnp.sum(oh0 * off, axis=1, keepdims=True)
    base1 = jnp.sum(oh1 * off, axis=1, keepdims=True)
    pos0 = (base0 + rank0).astype(jnp.int32)
    pos1 = (base1 + rank1).astype(jnp.int32)
    pos_ref[...] = jnp.concatenate([pos0, pos1], axis=1)

    Ieye = (li == lj).astype(jnp.float32)
    cumT = jax.lax.dot_general(
        Ieye, cum_incl, (((1,), (1,)), ((), ())),
        preferred_element_type=jnp.float32)
    bio = jax.lax.broadcasted_iota(jnp.int32, (E, 128), 1)
    owner = jnp.sum((cumT.astype(jnp.int32) <= bio).astype(jnp.float32),
                    axis=0, keepdims=True)
    be_ref[...] = owner.astype(jnp.int32)


_SC_CORES = 2
_SC_SUBCORES = 16
NWORK = _SC_CORES * _SC_SUBCORES
TPW = T // NWORK


def _sc_dispatch(x, pos0, pos1):
    mesh = plsc.VectorSubcoreMesh(core_axis_name="c", subcore_axis_name="s")

    @functools.partial(
        pl.kernel,
        out_type=jax.ShapeDtypeStruct((S_MAX, D), jnp.float32),
        mesh=mesh,
        scratch_types=[
            pltpu.VMEM((TPW,), jnp.int32),
            pltpu.VMEM((TPW,), jnp.int32),
            pltpu.VMEM((TPW, D), jnp.float32),
            pltpu.SemaphoreType.DMA,
        ],
    )
    def k(x_hbm, p0_hbm, p1_hbm, xs_hbm, idx0_v, idx1_v, rows_v, sem):
        wid = lax.axis_index("s") * _SC_CORES + lax.axis_index("c")
        base = wid * TPW
        pltpu.sync_copy(p0_hbm.at[pl.ds(base, TPW)], idx0_v)
        pltpu.sync_copy(p1_hbm.at[pl.ds(base, TPW)], idx1_v)
        pltpu.sync_copy(x_hbm.at[pl.ds(base, TPW)], rows_v)
        pltpu.async_copy(rows_v, xs_hbm.at[idx0_v], sem).wait()
        pltpu.async_copy(rows_v, xs_hbm.at[idx1_v], sem).wait()

    return k(x, pos0, pos1)


def _grouped_body(be_ref, xs_ref, wg0_ref, wu0_ref, wd0_ref,
                  wg1_ref, wu1_ref, wd1_ref, ys_ref):
    p = pl.program_id(0)

    def half(lo, wg_ref, wu_ref, wd_ref, sel):
        @pl.when(sel < E)
        def _():
            xs = xs_ref[pl.ds(lo, BR), :]
            g = jnp.dot(xs, wg_ref[0], preferred_element_type=jnp.float32)
            u = jnp.dot(xs, wu_ref[0], preferred_element_type=jnp.float32)
            h = _sigmoid(g) * g * u
            ys_ref[pl.ds(lo, BR), :] = jnp.dot(
                h, wd_ref[0], preferred_element_type=jnp.float32)

    half(0, wg0_ref, wu0_ref, wd0_ref, be_ref[2 * p])
    half(BR, wg1_ref, wu1_ref, wd1_ref, be_ref[2 * p + 1])


def _grouped(xs, be, Wg, Wu, Wd):
    def wmap(off):
        return lambda b, be: (jnp.minimum(be[2 * b + off], E - 1), 0, 0)

    grid_spec = pltpu.PrefetchScalarGridSpec(
        num_scalar_prefetch=1,
        grid=(NB // 2,),
        in_specs=[
            pl.BlockSpec((2 * BR, D), lambda b, be: (b, 0)),
            pl.BlockSpec((1, D, F), wmap(0)),
            pl.BlockSpec((1, D, F), wmap(0)),
            pl.BlockSpec((1, F, D), wmap(0)),
            pl.BlockSpec((1, D, F), wmap(1)),
            pl.BlockSpec((1, D, F), wmap(1)),
            pl.BlockSpec((1, F, D), wmap(1)),
        ],
        out_specs=pl.BlockSpec((2 * BR, D), lambda b, be: (b, 0)),
    )
    return pl.pallas_call(
        _grouped_body,
        grid_spec=grid_spec,
        out_shape=jax.ShapeDtypeStruct((S_MAX, D), jnp.float32),
        compiler_params=pltpu.CompilerParams(
            dimension_semantics=("arbitrary",)),
    )(be, xs, Wg, Wu, Wd, Wg, Wu, Wd)


def _sc_gather(ys, pos0, pos1):
    mesh = plsc.VectorSubcoreMesh(core_axis_name="c", subcore_axis_name="s")

    @functools.partial(
        pl.kernel,
        out_type=jax.ShapeDtypeStruct((2 * T, D), jnp.float32),
        mesh=mesh,
        scratch_types=[
            pltpu.VMEM((TPW,), jnp.int32),
            pltpu.VMEM((TPW, D), jnp.float32),
            pltpu.SemaphoreType.DMA,
        ],
    )
    def k(ys_hbm, p0_hbm, p1_hbm, yg_hbm, idx_v, rows_v, sem):
        wid = lax.axis_index("s") * _SC_CORES + lax.axis_index("c")
        base = wid * TPW
        pltpu.sync_copy(p0_hbm.at[pl.ds(base, TPW)], idx_v)
        pltpu.async_copy(ys_hbm.at[idx_v], rows_v, sem).wait()
        pltpu.sync_copy(rows_v, yg_hbm.at[pl.ds(base, TPW)])
        pltpu.sync_copy(p1_hbm.at[pl.ds(base, TPW)], idx_v)
        pltpu.async_copy(ys_hbm.at[idx_v], rows_v, sem).wait()
        pltpu.sync_copy(rows_v, yg_hbm.at[pl.ds(T + base, TPW)])

    return k(ys, pos0, pos1)


def _combine_body(sh_ref, yg0_ref, yg1_ref, ws_ref, o_ref):
    w0 = ws_ref[:, 0:1]
    w1 = ws_ref[:, 1:2]
    o_ref[...] = sh_ref[...] + w0 * yg0_ref[...] + w1 * yg1_ref[...]


def _combine(sh, yg, ws):
    return pl.pallas_call(
        _combine_body,
        grid=(T // BT,),
        in_specs=[
            pl.BlockSpec((BT, D), lambda i: (i, 0)),
            pl.BlockSpec((BT, D), lambda i: (i, 0)),
            pl.BlockSpec((BT, D), lambda i: (i + T // BT, 0)),
            pl.BlockSpec((BT, 2), lambda i: (i, 0)),
        ],
        out_specs=pl.BlockSpec((BT, D), lambda i: (i, 0)),
        out_shape=jax.ShapeDtypeStruct((T, D), jnp.float32),
    )(sh, yg, yg, ws)


def kernel(hidden_states, gate_w, e_score_correction_bias, Wg, Wu, Wd, Wsg, Wsu, Wsd):
    x = hidden_states
    bias = e_score_correction_bias.reshape(1, E)
    ids, ws, sh, pos, be = _routing(x, gate_w, bias, Wsg, Wsu, Wsd)
    pos0 = pos[:, 0]
    pos1 = pos[:, 1]
    be_vec = be[0, :]
    xs = _sc_dispatch(x, pos0, pos1)
    ys = _grouped(xs, be_vec, Wg, Wu, Wd)
    yg = _sc_gather(ys, pos0, pos1)
    return _combine(sh, yg, ws)


# speedup vs baseline: 1.2789x; 1.0562x over previous
import functools

import jax
import jax.numpy as jnp
from jax import lax
from jax.experimental import pallas as pl
from jax.experimental.pallas import tpu as pltpu
from jax.experimental.pallas import tpu_sc as plsc

T = 2048
D = 768
F = 256
E = 64
NG = 8
GS = E // NG
TG = 4
RSF = 1.0

NEG = -1e30
BT = 1024
BR = 128
NB = T * 2 // BR + E
S_MAX = NB * BR


def _sigmoid(x):
    return 1.0 / (1.0 + jnp.exp(-x))


def _routing_body(x_ref, gw_ref, b_ref, wsg_ref, wsu_ref, wsd_ref,
                  ids_ref, ws_ref, sh_ref, pos_ref, be_ref, idsc_ref):
    i = pl.program_id(0)

    @pl.when(i < T // BT)
    def _route():
        _route_step(x_ref, gw_ref, b_ref, wsg_ref, wsu_ref, wsd_ref,
                    ids_ref, ws_ref, sh_ref, idsc_ref, i)

    @pl.when(i == T // BT)
    def _do_plan():
        _plan_step(idsc_ref, pos_ref, be_ref)


def _route_step(x_ref, gw_ref, b_ref, wsg_ref, wsu_ref, wsd_ref,
                ids_ref, ws_ref, sh_ref, idsc_ref, i):
    x = x_ref[...]
    gw = gw_ref[...]
    logits = jax.lax.dot_general(
        x, gw, (((1,), (1,)), ((), ())), preferred_element_type=jnp.float32)
    scores = _sigmoid(logits)
    sfc = scores + b_ref[...]

    s3 = sfc.reshape(BT, NG, GS)
    gi = jax.lax.broadcasted_iota(jnp.int32, (BT, NG, GS), 2)
    m1 = jnp.max(s3, axis=-1)
    fidx = jnp.min(jnp.where(s3 >= m1[..., None], gi, GS), axis=-1)
    m2 = jnp.max(jnp.where(gi == fidx[..., None], NEG, s3), axis=-1)
    gsc = m1 + m2

    giota = jax.lax.broadcasted_iota(jnp.int32, (BT, NG), 1)
    gmask = jnp.zeros((BT, NG), jnp.float32)
    cur = gsc
    for _ in range(TG):
        mx = jnp.max(cur, axis=-1, keepdims=True)
        sel = jnp.min(jnp.where(cur >= mx, giota, NG), axis=-1, keepdims=True)
        hit = giota == sel
        gmask = jnp.where(hit, 1.0, gmask)
        cur = jnp.where(hit, NEG, cur)

    masked = jnp.where(gmask[..., None] > 0, s3, NEG).reshape(BT, E)
    eio = jax.lax.broadcasted_iota(jnp.int32, (BT, E), 1)
    mx1 = jnp.max(masked, axis=-1, keepdims=True)
    id1 = jnp.min(jnp.where(masked >= mx1, eio, E), axis=-1, keepdims=True)
    m2d = jnp.where(eio == id1, NEG, masked)
    mx2 = jnp.max(m2d, axis=-1, keepdims=True)
    id2 = jnp.min(jnp.where(m2d >= mx2, eio, E), axis=-1, keepdims=True)

    w1 = jnp.sum(jnp.where(eio == id1, scores, 0.0), axis=-1, keepdims=True)
    w2 = jnp.sum(jnp.where(eio == id2, scores, 0.0), axis=-1, keepdims=True)
    tot = w1 + w2 + 1e-20
    w1 = w1 / tot * RSF
    w2 = w2 / tot * RSF

    ids = jnp.concatenate([id1, id2], axis=1)
    ids_ref[...] = ids
    idsc_ref[pl.ds(i * BT, BT), :] = ids
    ws_ref[...] = jnp.concatenate([w1, w2], axis=1)

    gs = jnp.dot(x, wsg_ref[...], preferred_element_type=jnp.float32)
    us = jnp.dot(x, wsu_ref[...], preferred_element_type=jnp.float32)
    hs = _sigmoid(gs) * gs * us
    sh_ref[...] = jnp.dot(hs, wsd_ref[...], preferred_element_type=jnp.float32)


def _routing(x, gate_w, bias, wsg, wsu, wsd):
    nsteps = T // BT + 1
    return pl.pallas_call(
        _routing_body,
        grid=(nsteps,),
        in_specs=[
            pl.BlockSpec((BT, D), lambda i: (jnp.minimum(i, T // BT - 1), 0)),
            pl.BlockSpec((E, D), lambda i: (0, 0)),
            pl.BlockSpec((1, E), lambda i: (0, 0)),
            pl.BlockSpec((D, F), lambda i: (0, 0)),
            pl.BlockSpec((D, F), lambda i: (0, 0)),
            pl.BlockSpec((F, D), lambda i: (0, 0)),
        ],
        out_specs=[
            pl.BlockSpec((BT, 2), lambda i: (jnp.minimum(i, T // BT - 1), 0)),
            pl.BlockSpec((BT, 2), lambda i: (jnp.minimum(i, T // BT - 1), 0)),
            pl.BlockSpec((BT, D), lambda i: (jnp.minimum(i, T // BT - 1), 0)),
            pl.BlockSpec((T, 2), lambda i: (0, 0)),
            pl.BlockSpec((1, 128), lambda i: (0, 0)),
        ],
        out_shape=[
            jax.ShapeDtypeStruct((T, 2), jnp.int32),
            jax.ShapeDtypeStruct((T, 2), jnp.float32),
            jax.ShapeDtypeStruct((T, D), jnp.float32),
            jax.ShapeDtypeStruct((T, 2), jnp.int32),
            jax.ShapeDtypeStruct((1, 128), jnp.int32),
        ],
        scratch_shapes=[pltpu.VMEM((T, 2), jnp.int32)],
        compiler_params=pltpu.CompilerParams(
            dimension_semantics=("arbitrary",)),
    )(x, gate_w, bias, wsg, wsu, wsd)


CH = 512


def _plan_step(ids_ref, pos_ref, be_ref):
    ids = ids_ref[...]
    e0 = ids[:, 0:1]
    e1 = ids[:, 1:2]
    lane = jax.lax.broadcasted_iota(jnp.int32, (T, E), 1)
    oh0 = (lane == e0).astype(jnp.float32)
    oh1 = (lane == e1).astype(jnp.float32)

    ri = jax.lax.broadcasted_iota(jnp.int32, (CH, CH), 0)
    ci = jax.lax.broadcasted_iota(jnp.int32, (CH, CH), 1)
    Ls = (ci < ri).astype(jnp.float32)

    def prefix_excl(oh):
        outs = []
        carry = jnp.zeros((1, E), jnp.float32)
        for c in range(T // CH):
            blk = oh[c * CH:(c + 1) * CH, :]
            p = jax.lax.dot_general(
                Ls, blk, (((1,), (0,)), ((), ())),
                preferred_element_type=jnp.float32) + carry
            carry = carry + jnp.sum(blk, axis=0, keepdims=True)
            outs.append(p)
        return jnp.concatenate(outs, axis=0), carry

    P0, cnt0 = prefix_excl(oh0)
    P1, cnt1 = prefix_excl(oh1)
    P1 = P1 + cnt0
    counts = cnt0 + cnt1

    nb = jnp.floor((counts + (BR - 1)) / BR)
    li = jax.lax.broadcasted_iota(jnp.int32, (E, E), 0)
    lj = jax.lax.broadcasted_iota(jnp.int32, (E, E), 1)
    Uincl = (li <= lj).astype(jnp.float32)
    Uexcl = (li < lj).astype(jnp.float32)
    cum_incl = jnp.dot(nb, Uincl, preferred_element_type=jnp.float32)
    cum_excl = jnp.dot(nb, Uexcl, preferred_element_type=jnp.float32)
    off = cum_excl * BR

    rank0 = jnp.sum(oh0 * P0, axis=1, keepdims=True)
    rank1 = jnp.sum(oh1 * P1, axis=1, keepdims=True)
    base0 = jnp.sum(oh0 * off, axis=1, keepdims=True)
    base1 = jnp.sum(oh1 * off, axis=1, keepdims=True)
    pos0 = (base0 + rank0).astype(jnp.int32)
    pos1 = (base1 + rank1).astype(jnp.int32)
    pos_ref[...] = jnp.concatenate([pos0, pos1], axis=1)

    Ieye = (li == lj).astype(jnp.float32)
    cumT = jax.lax.dot_general(
        Ieye, cum_incl, (((1,), (1,)), ((), ())),
        preferred_element_type=jnp.float32)
    bio = jax.lax.broadcasted_iota(jnp.int32, (E, 128), 1)
    owner = jnp.sum((cumT.astype(jnp.int32) <= bio).astype(jnp.float32),
                    axis=0, keepdims=True)
    be_ref[...] = owner.astype(jnp.int32)


_SC_CORES = 2
_SC_SUBCORES = 16
NWORK = _SC_CORES * _SC_SUBCORES
TPW = T // NWORK


def _sc_dispatch(x, pos0, pos1):
    mesh = plsc.VectorSubcoreMesh(core_axis_name="c", subcore_axis_name="s")

    @functools.partial(
        pl.kernel,
        out_type=jax.ShapeDtypeStruct((S_MAX, D), jnp.float32),
        mesh=mesh,
        scratch_types=[
            pltpu.VMEM((TPW,), jnp.int32),
            pltpu.VMEM((TPW,), jnp.int32),
            pltpu.VMEM((TPW, D), jnp.float32),
            pltpu.SemaphoreType.DMA,
        ],
    )
    def k(x_hbm, p0_hbm, p1_hbm, xs_hbm, idx0_v, idx1_v, rows_v, sem):
        wid = lax.axis_index("s") * _SC_CORES + lax.axis_index("c")
        base = wid * TPW
        pltpu.sync_copy(p0_hbm.at[pl.ds(base, TPW)], idx0_v)
        pltpu.sync_copy(p1_hbm.at[pl.ds(base, TPW)], idx1_v)
        pltpu.sync_copy(x_hbm.at[pl.ds(base, TPW)], rows_v)
        pltpu.async_copy(rows_v, xs_hbm.at[idx0_v], sem).wait()
        pltpu.async_copy(rows_v, xs_hbm.at[idx1_v], sem).wait()

    return k(x, pos0, pos1)


PB = 4


def _grouped_body(be_ref, xs_ref, *rest):
    wrefs = rest[:-1]
    ys_ref = rest[-1]
    p = pl.program_id(0)

    def part(j):
        @pl.when(be_ref[PB * p + j] < E)
        def _():
            xs = xs_ref[pl.ds(j * BR, BR), :]
            wg_ref, wu_ref, wd_ref = wrefs[3 * j:3 * j + 3]
            g = jnp.dot(xs, wg_ref[0], preferred_element_type=jnp.float32)
            u = jnp.dot(xs, wu_ref[0], preferred_element_type=jnp.float32)
            h = _sigmoid(g) * g * u
            ys_ref[pl.ds(j * BR, BR), :] = jnp.dot(
                h, wd_ref[0], preferred_element_type=jnp.float32)

    for j in range(PB):
        part(j)


def _grouped(xs, be, Wg, Wu, Wd):
    def wmap(off):
        return lambda b, be: (jnp.minimum(be[PB * b + off], E - 1), 0, 0)

    wspecs = []
    for j in range(PB):
        wspecs += [
            pl.BlockSpec((1, D, F), wmap(j)),
            pl.BlockSpec((1, D, F), wmap(j)),
            pl.BlockSpec((1, F, D), wmap(j)),
        ]
    grid_spec = pltpu.PrefetchScalarGridSpec(
        num_scalar_prefetch=1,
        grid=(NB // PB,),
        in_specs=[pl.BlockSpec((PB * BR, D), lambda b, be: (b, 0))] + wspecs,
        out_specs=pl.BlockSpec((PB * BR, D), lambda b, be: (b, 0)),
    )
    return pl.pallas_call(
        _grouped_body,
        grid_spec=grid_spec,
        out_shape=jax.ShapeDtypeStruct((S_MAX, D), jnp.float32),
        compiler_params=pltpu.CompilerParams(
            dimension_semantics=("arbitrary",)),
    )(be, xs, *([Wg, Wu, Wd] * PB))


def _sc_gather(ys, pos0, pos1):
    mesh = plsc.VectorSubcoreMesh(core_axis_name="c", subcore_axis_name="s")

    @functools.partial(
        pl.kernel,
        out_type=jax.ShapeDtypeStruct((2 * T, D), jnp.float32),
        mesh=mesh,
        scratch_types=[
            pltpu.VMEM((TPW,), jnp.int32),
            pltpu.VMEM((TPW, D), jnp.float32),
            pltpu.SemaphoreType.DMA,
        ],
    )
    def k(ys_hbm, p0_hbm, p1_hbm, yg_hbm, idx_v, rows_v, sem):
        wid = lax.axis_index("s") * _SC_CORES + lax.axis_index("c")
        base = wid * TPW
        pltpu.sync_copy(p0_hbm.at[pl.ds(base, TPW)], idx_v)
        pltpu.async_copy(ys_hbm.at[idx_v], rows_v, sem).wait()
        pltpu.sync_copy(rows_v, yg_hbm.at[pl.ds(base, TPW)])
        pltpu.sync_copy(p1_hbm.at[pl.ds(base, TPW)], idx_v)
        pltpu.async_copy(ys_hbm.at[idx_v], rows_v, sem).wait()
        pltpu.sync_copy(rows_v, yg_hbm.at[pl.ds(T + base, TPW)])

    return k(ys, pos0, pos1)


def _combine_body(sh_ref, yg0_ref, yg1_ref, ws_ref, o_ref):
    w0 = ws_ref[:, 0:1]
    w1 = ws_ref[:, 1:2]
    o_ref[...] = sh_ref[...] + w0 * yg0_ref[...] + w1 * yg1_ref[...]


def _combine(sh, yg, ws):
    return pl.pallas_call(
        _combine_body,
        grid=(T // BT,),
        in_specs=[
            pl.BlockSpec((BT, D), lambda i: (i, 0)),
            pl.BlockSpec((BT, D), lambda i: (i, 0)),
            pl.BlockSpec((BT, D), lambda i: (i + T // BT, 0)),
            pl.BlockSpec((BT, 2), lambda i: (i, 0)),
        ],
        out_specs=pl.BlockSpec((BT, D), lambda i: (i, 0)),
        out_shape=jax.ShapeDtypeStruct((T, D), jnp.float32),
    )(sh, yg, yg, ws)


def kernel(hidden_states, gate_w, e_score_correction_bias, Wg, Wu, Wd, Wsg, Wsu, Wsd):
    x = hidden_states
    bias = e_score_correction_bias.reshape(1, E)
    ids, ws, sh, pos, be = _routing(x, gate_w, bias, Wsg, Wsu, Wsd)
    pos0 = pos[:, 0]
    pos1 = pos[:, 1]
    be_vec = be[0, :]
    xs = _sc_dispatch(x, pos0, pos1)
    ys = _grouped(xs, be_vec, Wg, Wu, Wd)
    yg = _sc_gather(ys, pos0, pos1)
    return _combine(sh, yg, ws)
